# baseline (device time: 53676 ns/iter reference)
import jax
import jax.numpy as jnp
from jax import lax
from jax.experimental import pallas as pl
from jax.experimental.pallas import tpu as pltpu


def kernel(x, pi):
    b, m, n = x.shape

    def body(x_ref, pi_ref, out_ref, send_sem, recv_sem):
        my_x = lax.axis_index("x")
        my_y = lax.axis_index("y")
        dest_x = pi_ref[my_x]
        other_x = 1 - my_x

        barrier_sem = pltpu.get_barrier_semaphore()
        pl.semaphore_signal(
            barrier_sem,
            inc=1,
            device_id=(other_x, my_y),
            device_id_type=pl.DeviceIdType.MESH,
        )
        pl.semaphore_wait(barrier_sem, 1)

        @pl.when(dest_x == my_x)
        def _():
            out_ref[...] = x_ref[...]

        @pl.when(dest_x != my_x)
        def _():
            rdma = pltpu.make_async_remote_copy(
                src_ref=x_ref,
                dst_ref=out_ref,
                send_sem=send_sem,
                recv_sem=recv_sem,
                device_id=(dest_x, my_y),
                device_id_type=pl.DeviceIdType.MESH,
            )
            rdma.start()
            rdma.wait()

    return pl.pallas_call(
        body,
        out_shape=jax.ShapeDtypeStruct((b, m, n), jnp.float32),
        in_specs=[
            pl.BlockSpec(memory_space=pltpu.VMEM),
            pl.BlockSpec(memory_space=pltpu.SMEM),
        ],
        out_specs=pl.BlockSpec(memory_space=pltpu.VMEM),
        scratch_shapes=[
            pltpu.SemaphoreType.DMA,
            pltpu.SemaphoreType.DMA,
        ],
        compiler_params=pltpu.CompilerParams(collective_id=0),
    )(x, pi)


# device time: 53651 ns/iter; 1.0005x vs baseline; 1.0005x over previous
import jax
import jax.numpy as jnp
from jax import lax
from jax.experimental import pallas as pl
from jax.experimental.pallas import tpu as pltpu


def kernel(x, pi):
    b, m, n = x.shape

    def body(x_ref, pi_ref, out_ref, send_sem, recv_sem):
        my_x = lax.axis_index("x")
        my_y = lax.axis_index("y")
        dest_x = pi_ref[my_x]
        other_x = 1 - my_x

        barrier_sem = pltpu.get_barrier_semaphore()
        pl.semaphore_signal(
            barrier_sem,
            inc=1,
            device_id=(other_x, my_y),
            device_id_type=pl.DeviceIdType.MESH,
        )
        pl.semaphore_wait(barrier_sem, 1)

        @pl.when(dest_x == my_x)
        def _():
            copy = pltpu.make_async_copy(x_ref, out_ref, send_sem)
            copy.start()
            copy.wait()

        @pl.when(dest_x != my_x)
        def _():
            rdma = pltpu.make_async_remote_copy(
                src_ref=x_ref,
                dst_ref=out_ref,
                send_sem=send_sem,
                recv_sem=recv_sem,
                device_id=(dest_x, my_y),
                device_id_type=pl.DeviceIdType.MESH,
            )
            rdma.start()
            rdma.wait()

    return pl.pallas_call(
        body,
        out_shape=jax.ShapeDtypeStruct((b, m, n), jnp.float32),
        in_specs=[
            pl.BlockSpec(memory_space=pl.ANY),
            pl.BlockSpec(memory_space=pltpu.SMEM),
        ],
        out_specs=pl.BlockSpec(memory_space=pl.ANY),
        scratch_shapes=[
            pltpu.SemaphoreType.DMA,
            pltpu.SemaphoreType.DMA,
        ],
        compiler_params=pltpu.CompilerParams(collective_id=0),
    )(x, pi)


# device time: 36740 ns/iter; 1.4610x vs baseline; 1.4603x over previous
import jax
import jax.numpy as jnp
from jax import lax
from jax.experimental import pallas as pl
from jax.experimental.pallas import tpu as pltpu

_N_CHUNK = 8


def kernel(x, pi):
    b, m, n = x.shape
    half = m // 2
    rpc = half // _N_CHUNK

    def body(x_ref, pi_ref, out_ref, sx_sems, sy_sems, rx_sems, ry_sems,
             local_sem):
        my_x = lax.axis_index("x")
        my_y = lax.axis_index("y")
        dest_x = pi_ref[my_x]
        other_x = 1 - my_x
        other_y = 1 - my_y

        barrier_sem = pltpu.get_barrier_semaphore()
        for nbr in ((other_x, my_y), (my_x, other_y)):
            pl.semaphore_signal(
                barrier_sem, inc=1,
                device_id=nbr, device_id_type=pl.DeviceIdType.MESH,
            )
        pl.semaphore_wait(barrier_sem, 2)

        @pl.when(dest_x == my_x)
        def _():
            copy = pltpu.make_async_copy(x_ref, out_ref, local_sem)
            copy.start()
            copy.wait()

        @pl.when(dest_x != my_x)
        def _():
            my_half = my_y * half

            x_sends = []
            for c in range(_N_CHUNK):
                sl = pl.ds(my_half + c * rpc, rpc)
                rdma = pltpu.make_async_remote_copy(
                    src_ref=x_ref.at[:, sl, :],
                    dst_ref=out_ref.at[:, sl, :],
                    send_sem=sx_sems.at[c],
                    recv_sem=rx_sems.at[c],
                    device_id=(dest_x, my_y),
                    device_id_type=pl.DeviceIdType.MESH,
                )
                rdma.start()
                x_sends.append(rdma)

            y_sends = []
            for c in range(_N_CHUNK):
                sl = pl.ds(my_half + c * rpc, rpc)
                recv = pltpu.make_async_remote_copy(
                    src_ref=out_ref.at[:, sl, :],
                    dst_ref=out_ref.at[:, sl, :],
                    send_sem=sx_sems.at[c],
                    recv_sem=rx_sems.at[c],
                    device_id=(other_x, my_y),
                    device_id_type=pl.DeviceIdType.MESH,
                )
                recv.wait_recv()
                fwd = pltpu.make_async_remote_copy(
                    src_ref=out_ref.at[:, sl, :],
                    dst_ref=out_ref.at[:, sl, :],
                    send_sem=sy_sems.at[c],
                    recv_sem=ry_sems.at[c],
                    device_id=(my_x, other_y),
                    device_id_type=pl.DeviceIdType.MESH,
                )
                fwd.start()
                y_sends.append(fwd)

            for c in range(_N_CHUNK):
                sl = pl.ds(other_y * half + c * rpc, rpc)
                recv = pltpu.make_async_remote_copy(
                    src_ref=out_ref.at[:, sl, :],
                    dst_ref=out_ref.at[:, sl, :],
                    send_sem=sy_sems.at[c],
                    recv_sem=ry_sems.at[c],
                    device_id=(my_x, other_y),
                    device_id_type=pl.DeviceIdType.MESH,
                )
                recv.wait_recv()

            for rdma in x_sends:
                rdma.wait_send()
            for rdma in y_sends:
                rdma.wait_send()

    return pl.pallas_call(
        body,
        out_shape=jax.ShapeDtypeStruct((b, m, n), jnp.float32),
        in_specs=[
            pl.BlockSpec(memory_space=pl.ANY),
            pl.BlockSpec(memory_space=pltpu.SMEM),
        ],
        out_specs=pl.BlockSpec(memory_space=pl.ANY),
        scratch_shapes=[
            pltpu.SemaphoreType.DMA((_N_CHUNK,)),
            pltpu.SemaphoreType.DMA((_N_CHUNK,)),
            pltpu.SemaphoreType.DMA((_N_CHUNK,)),
            pltpu.SemaphoreType.DMA((_N_CHUNK,)),
            pltpu.SemaphoreType.DMA,
        ],
        compiler_params=pltpu.CompilerParams(collective_id=0),
    )(x, pi)


# device time: 35617 ns/iter; 1.5070x vs baseline; 1.0315x over previous
import jax
import jax.numpy as jnp
from jax import lax
from jax.experimental import pallas as pl
from jax.experimental.pallas import tpu as pltpu

_N_CHUNK = 16


def kernel(x, pi):
    b, m, n = x.shape
    half = m // 2
    rpc = half // _N_CHUNK

    def body(x_ref, pi_ref, out_ref, sx_sems, sy_sems, rx_sems, ry_sems,
             local_sem):
        my_x = lax.axis_index("x")
        my_y = lax.axis_index("y")
        dest_x = pi_ref[my_x]
        other_x = 1 - my_x
        other_y = 1 - my_y

        barrier_sem = pltpu.get_barrier_semaphore()
        for nbr in ((other_x, my_y), (my_x, other_y)):
            pl.semaphore_signal(
                barrier_sem, inc=1,
                device_id=nbr, device_id_type=pl.DeviceIdType.MESH,
            )
        pl.semaphore_wait(barrier_sem, 2)

        @pl.when(dest_x == my_x)
        def _():
            copy = pltpu.make_async_copy(x_ref, out_ref, local_sem)
            copy.start()
            copy.wait()

        @pl.when(dest_x != my_x)
        def _():
            my_half = my_y * half

            x_sends = []
            for c in range(_N_CHUNK):
                sl = pl.ds(my_half + c * rpc, rpc)
                rdma = pltpu.make_async_remote_copy(
                    src_ref=x_ref.at[:, sl, :],
                    dst_ref=out_ref.at[:, sl, :],
                    send_sem=sx_sems.at[c],
                    recv_sem=rx_sems.at[c],
                    device_id=(dest_x, my_y),
                    device_id_type=pl.DeviceIdType.MESH,
                )
                rdma.start()
                x_sends.append(rdma)

            y_sends = []
            for c in range(_N_CHUNK):
                sl = pl.ds(my_half + c * rpc, rpc)
                recv = pltpu.make_async_remote_copy(
                    src_ref=out_ref.at[:, sl, :],
                    dst_ref=out_ref.at[:, sl, :],
                    send_sem=sx_sems.at[c],
                    recv_sem=rx_sems.at[c],
                    device_id=(other_x, my_y),
                    device_id_type=pl.DeviceIdType.MESH,
                )
                recv.wait_recv()
                fwd = pltpu.make_async_remote_copy(
                    src_ref=out_ref.at[:, sl, :],
                    dst_ref=out_ref.at[:, sl, :],
                    send_sem=sy_sems.at[c],
                    recv_sem=ry_sems.at[c],
                    device_id=(my_x, other_y),
                    device_id_type=pl.DeviceIdType.MESH,
                )
                fwd.start()
                y_sends.append(fwd)

            for c in range(_N_CHUNK):
                sl = pl.ds(other_y * half + c * rpc, rpc)
                recv = pltpu.make_async_remote_copy(
                    src_ref=out_ref.at[:, sl, :],
                    dst_ref=out_ref.at[:, sl, :],
                    send_sem=sy_sems.at[c],
                    recv_sem=ry_sems.at[c],
                    device_id=(my_x, other_y),
                    device_id_type=pl.DeviceIdType.MESH,
                )
                recv.wait_recv()

            for rdma in x_sends:
                rdma.wait_send()
            for rdma in y_sends:
                rdma.wait_send()

    return pl.pallas_call(
        body,
        out_shape=jax.ShapeDtypeStruct((b, m, n), jnp.float32),
        in_specs=[
            pl.BlockSpec(memory_space=pl.ANY),
            pl.BlockSpec(memory_space=pltpu.SMEM),
        ],
        out_specs=pl.BlockSpec(memory_space=pl.ANY),
        scratch_shapes=[
            pltpu.SemaphoreType.DMA((_N_CHUNK,)),
            pltpu.SemaphoreType.DMA((_N_CHUNK,)),
            pltpu.SemaphoreType.DMA((_N_CHUNK,)),
            pltpu.SemaphoreType.DMA((_N_CHUNK,)),
            pltpu.SemaphoreType.DMA,
        ],
        compiler_params=pltpu.CompilerParams(collective_id=0),
    )(x, pi)


# device time: 35518 ns/iter; 1.5112x vs baseline; 1.0028x over previous
import jax
import jax.numpy as jnp
from jax import lax
from jax.experimental import pallas as pl
from jax.experimental.pallas import tpu as pltpu

_N_CHUNK = 32


def kernel(x, pi):
    b, m, n = x.shape
    half = m // 2
    rpc = half // _N_CHUNK

    def body(x_ref, pi_ref, out_ref, sx_sems, sy_sems, rx_sems, ry_sems,
             local_sem):
        my_x = lax.axis_index("x")
        my_y = lax.axis_index("y")
        dest_x = pi_ref[my_x]
        other_x = 1 - my_x
        other_y = 1 - my_y

        barrier_sem = pltpu.get_barrier_semaphore()
        for nbr in ((other_x, my_y), (my_x, other_y)):
            pl.semaphore_signal(
                barrier_sem, inc=1,
                device_id=nbr, device_id_type=pl.DeviceIdType.MESH,
            )
        pl.semaphore_wait(barrier_sem, 2)

        @pl.when(dest_x == my_x)
        def _():
            copy = pltpu.make_async_copy(x_ref, out_ref, local_sem)
            copy.start()
            copy.wait()

        @pl.when(dest_x != my_x)
        def _():
            my_half = my_y * half

            x_sends = []
            for c in range(_N_CHUNK):
                sl = pl.ds(my_half + c * rpc, rpc)
                rdma = pltpu.make_async_remote_copy(
                    src_ref=x_ref.at[:, sl, :],
                    dst_ref=out_ref.at[:, sl, :],
                    send_sem=sx_sems.at[c],
                    recv_sem=rx_sems.at[c],
                    device_id=(dest_x, my_y),
                    device_id_type=pl.DeviceIdType.MESH,
                )
                rdma.start()
                x_sends.append(rdma)

            y_sends = []
            for c in range(_N_CHUNK):
                sl = pl.ds(my_half + c * rpc, rpc)
                recv = pltpu.make_async_remote_copy(
                    src_ref=out_ref.at[:, sl, :],
                    dst_ref=out_ref.at[:, sl, :],
                    send_sem=sx_sems.at[c],
                    recv_sem=rx_sems.at[c],
                    device_id=(other_x, my_y),
                    device_id_type=pl.DeviceIdType.MESH,
                )
                recv.wait_recv()
                fwd = pltpu.make_async_remote_copy(
                    src_ref=out_ref.at[:, sl, :],
                    dst_ref=out_ref.at[:, sl, :],
                    send_sem=sy_sems.at[c],
                    recv_sem=ry_sems.at[c],
                    device_id=(my_x, other_y),
                    device_id_type=pl.DeviceIdType.MESH,
                )
                fwd.start()
                y_sends.append(fwd)

            for c in range(_N_CHUNK):
                sl = pl.ds(other_y * half + c * rpc, rpc)
                recv = pltpu.make_async_remote_copy(
                    src_ref=out_ref.at[:, sl, :],
                    dst_ref=out_ref.at[:, sl, :],
                    send_sem=sy_sems.at[c],
                    recv_sem=ry_sems.at[c],
                    device_id=(my_x, other_y),
                    device_id_type=pl.DeviceIdType.MESH,
                )
                recv.wait_recv()

            for rdma in x_sends:
                rdma.wait_send()
            for rdma in y_sends:
                rdma.wait_send()

    return pl.pallas_call(
        body,
        out_shape=jax.ShapeDtypeStruct((b, m, n), jnp.float32),
        in_specs=[
            pl.BlockSpec(memory_space=pl.ANY),
            pl.BlockSpec(memory_space=pltpu.SMEM),
        ],
        out_specs=pl.BlockSpec(memory_space=pl.ANY),
        scratch_shapes=[
            pltpu.SemaphoreType.DMA((_N_CHUNK,)),
            pltpu.SemaphoreType.DMA((_N_CHUNK,)),
            pltpu.SemaphoreType.DMA((_N_CHUNK,)),
            pltpu.SemaphoreType.DMA((_N_CHUNK,)),
            pltpu.SemaphoreType.DMA,
        ],
        compiler_params=pltpu.CompilerParams(collective_id=0),
    )(x, pi)
